# Initial kernel scaffold; baseline (speedup 1.0000x reference)
import jax
import jax.numpy as jnp
from jax.experimental import pallas as pl


def kernel(nf_gc, nf_gs, ei_s2c, ei_c2s, lin_gc_w, lin_gc_b, lin_gs_w, lin_gs_b, msg_s2c_w1, msg_s2c_b1, msg_s2c_w2, msg_s2c_b2, red_s2c_w, red_s2c_b, msg_c2s_w1, msg_c2s_b1, msg_c2s_w2, msg_c2s_b2, red_c2s_w, red_c2s_b, gc_w1, gc_b1, gc_w2, gc_b2, gs_w1, gs_b1, gs_w2, gs_b2):
    raise NotImplementedError("write your pallas kernel here")


# speedup vs baseline: 1.0241x; 1.0241x over previous
import functools

import jax
import jax.numpy as jnp
from jax.experimental import pallas as pl
from jax.experimental.pallas import tpu as pltpu

N_HID = 128


def _node_proj_body(nf_gc_ref, nf_gs_ref, wgc_ref, bgc_ref, wgs_ref, bgs_ref,
                    gc_ref, gs_ref):
    gc_ref[...] = nf_gc_ref[...] @ wgc_ref[...] + bgc_ref[...]
    gs_ref[...] = nf_gs_ref[...] @ wgs_ref[...] + bgs_ref[...]


def _node_proj(nf_gc, nf_gs, wgc, bgc, wgs, bgs):
    n_gc, _ = nf_gc.shape
    n_gs, _ = nf_gs.shape
    return pl.pallas_call(
        _node_proj_body,
        out_shape=(
            jax.ShapeDtypeStruct((n_gc, N_HID), jnp.float32),
            jax.ShapeDtypeStruct((n_gs, N_HID), jnp.float32),
        ),
    )(nf_gc, nf_gs, wgc, bgc.reshape(1, -1), wgs, bgs.reshape(1, -1))


def _mlp3(x, w1, b1, w2, b2):
    return jax.nn.relu(x @ w1 + b1) @ w2 + b2


def _msg_agg(src_x, dst_x, ei, w1, b1, w2, b2):
    xs = jnp.take(src_x, ei[0], axis=0)
    xd = jnp.take(dst_x, ei[1], axis=0)
    h = _mlp3(jnp.concatenate([xs, xd], axis=1), w1, b1, w2, b2)
    k = jax.nn.sigmoid(h[:, :1])
    f1 = h[:, 1:1 + N_HID] * k
    f2 = h[:, 1 + N_HID:1 + 2 * N_HID] * k
    f3 = h[:, 1 + 2 * N_HID:1 + 3 * N_HID] * k
    f4 = h[:, 1 + 3 * N_HID:1 + 4 * N_HID] * k
    nseg = dst_x.shape[0]
    a1 = jax.ops.segment_sum(f1, ei[1], num_segments=nseg)
    a2 = jax.ops.segment_max(f2, ei[1], num_segments=nseg)
    a3 = jax.ops.segment_min(f3, ei[1], num_segments=nseg)
    a4 = jax.ops.segment_sum(f4, ei[1], num_segments=nseg)
    return a1, a2, a3, a4


def kernel(nf_gc, nf_gs, ei_s2c, ei_c2s, lin_gc_w, lin_gc_b, lin_gs_w, lin_gs_b,
           msg_s2c_w1, msg_s2c_b1, msg_s2c_w2, msg_s2c_b2,
           red_s2c_w, red_s2c_b,
           msg_c2s_w1, msg_c2s_b1, msg_c2s_w2, msg_c2s_b2,
           red_c2s_w, red_c2s_b,
           gc_w1, gc_b1, gc_w2, gc_b2,
           gs_w1, gs_b1, gs_w2, gs_b2):
    gc_x, gs_x = _node_proj(nf_gc, nf_gs, lin_gc_w, lin_gc_b, lin_gs_w, lin_gs_b)
    a1, a2, a3, a4 = _msg_agg(gs_x, gc_x, ei_s2c, msg_s2c_w1, msg_s2c_b1,
                              msg_s2c_w2, msg_s2c_b2)
    new_cx = jnp.concatenate([gc_x, a1, a2, a3, a4], axis=1) @ red_s2c_w + red_s2c_b
    c1, c2, c3, c4 = _msg_agg(gc_x, gs_x, ei_c2s, msg_c2s_w1, msg_c2s_b1,
                              msg_c2s_w2, msg_c2s_b2)
    new_sx = jnp.concatenate([gs_x, c1, c2, c3, c4], axis=1) @ red_c2s_w + red_c2s_b
    out_fc = _mlp3(jnp.concatenate([gc_x, new_cx], axis=1), gc_w1, gc_b1, gc_w2, gc_b2)
    out_fs = _mlp3(jnp.concatenate([gs_x, new_sx], axis=1), gs_w1, gs_b1, gs_w2, gs_b2)
    return (out_fc, out_fs)


# speedup vs baseline: 1.4023x; 1.3693x over previous
import functools

import jax
import jax.numpy as jnp
from jax import lax
from jax.experimental import pallas as pl
from jax.experimental.pallas import tpu as pltpu
from jax.experimental.pallas import tpu_sc as plsc

N_NODE = 10000
E_TOT = 320000
H = 128


def _stage_a_body(nf_gc, nf_gs, wgc, bgc, wgs, bgs, w1s2c, b1s2c, w1c2s, b1c2s,
                  gc_x, gs_x, ps_s2c, pd_s2c, ps_c2s, pd_c2s):
    xc = nf_gc[...] @ wgc[...] + bgc[...]
    xs = nf_gs[...] @ wgs[...] + bgs[...]
    gc_x[...] = xc
    gs_x[...] = xs
    ps_s2c[...] = xs @ w1s2c[0:H, :]
    pd_s2c[...] = xc @ w1s2c[H:2 * H, :] + b1s2c[...]
    ps_c2s[...] = xc @ w1c2s[0:H, :]
    pd_c2s[...] = xs @ w1c2s[H:2 * H, :] + b1c2s[...]


def _stage_a(nf_gc, nf_gs, wgc, bgc, wgs, bgs, w1s2c, b1s2c, w1c2s, b1c2s):
    n = nf_gc.shape[0]
    blk = 1000
    grid = n // blk
    row_spec = pl.BlockSpec((blk, H), lambda i: (i, 0))
    out_spec = pl.BlockSpec((blk, 2 * H), lambda i: (i, 0))
    full = lambda shape: pl.BlockSpec(shape, lambda i: tuple(0 for _ in shape))
    return pl.pallas_call(
        _stage_a_body,
        grid=(grid,),
        in_specs=[row_spec, row_spec,
                  full((H, H)), full((1, H)), full((H, H)), full((1, H)),
                  full((2 * H, 2 * H)), full((1, 2 * H)),
                  full((2 * H, 2 * H)), full((1, 2 * H))],
        out_specs=(row_spec, row_spec, out_spec, out_spec, out_spec, out_spec),
        out_shape=(
            jax.ShapeDtypeStruct((n, H), jnp.float32),
            jax.ShapeDtypeStruct((n, H), jnp.float32),
            jax.ShapeDtypeStruct((n, 2 * H), jnp.float32),
            jax.ShapeDtypeStruct((n, 2 * H), jnp.float32),
            jax.ShapeDtypeStruct((n, 2 * H), jnp.float32),
            jax.ShapeDtypeStruct((n, 2 * H), jnp.float32),
        ),
    )(nf_gc, nf_gs, wgc, bgc.reshape(1, -1), wgs, bgs.reshape(1, -1),
      w1s2c, b1s2c.reshape(1, -1), w1c2s, b1c2s.reshape(1, -1))


def _prep_body(w2_1, w2_4, wr1, wr4, b2_1, b2_4, wm, bm):
    wm[...] = w2_1[...] @ wr1[...] + w2_4[...] @ wr4[...]
    bm[...] = b2_1[...] @ wr1[...] + b2_4[...] @ wr4[...]


def _prep(w2, b2, red_w):
    wm, bm = pl.pallas_call(
        _prep_body,
        out_shape=(jax.ShapeDtypeStruct((2 * H, H), jnp.float32),
                   jax.ShapeDtypeStruct((1, H), jnp.float32)),
    )(w2[:, 1:1 + H], w2[:, 1 + 3 * H:1 + 4 * H],
      red_w[H:2 * H], red_w[4 * H:5 * H],
      b2[1:1 + H].reshape(1, H), b2[1 + 3 * H:1 + 4 * H].reshape(1, H))
    aux_w = jnp.pad(w2[:, 0:1], ((0, 0), (0, H - 1)))
    aux_b = jnp.pad(b2[0:1], (0, H - 1)).reshape(1, H)
    w2cat = jnp.concatenate(
        [wm, w2[:, 1 + H:1 + 2 * H], w2[:, 1 + 2 * H:1 + 3 * H], aux_w], axis=1)
    b2cat = jnp.concatenate(
        [bm, b2[1 + H:1 + 2 * H].reshape(1, H),
         b2[1 + 2 * H:1 + 3 * H].reshape(1, H), aux_b], axis=1)
    return w2cat, b2cat


_E_PER_W = E_TOT // 32
_CH = 80
_NCH = _E_PER_W // _CH


def _gather_body(ps, pd, src, dst, ea, eb, idxs, idxd, bufa, bufb, sem1, sem2):
    info = plsc.get_sparse_core_info()
    nc = info.num_cores
    wid = lax.axis_index("s") * nc + lax.axis_index("c")
    base = wid * _E_PER_W

    def step(c, carry):
        off = base + c * _CH
        pltpu.sync_copy(src.at[pl.ds(off, _CH)], idxs)
        pltpu.sync_copy(dst.at[pl.ds(off, _CH)], idxd)
        cp1 = pltpu.async_copy(ps.at[idxs], bufa, sem1)
        cp2 = pltpu.async_copy(pd.at[idxd], bufb, sem2)
        cp1.wait()
        cp2.wait()
        pltpu.sync_copy(bufa, ea.at[pl.ds(off, _CH)])
        pltpu.sync_copy(bufb, eb.at[pl.ds(off, _CH)])
        return carry

    lax.fori_loop(0, _NCH, step, 0)


def _sc_gather(ps, pd, src, dst):
    mesh = plsc.VectorSubcoreMesh(core_axis_name="c", subcore_axis_name="s")
    f = pl.kernel(
        _gather_body,
        out_type=(jax.ShapeDtypeStruct((E_TOT, 2 * H), jnp.float32),
                  jax.ShapeDtypeStruct((E_TOT, 2 * H), jnp.float32)),
        mesh=mesh,
        scratch_types=[
            pltpu.VMEM((_CH,), jnp.int32),
            pltpu.VMEM((_CH,), jnp.int32),
            pltpu.VMEM((_CH, 2 * H), jnp.float32),
            pltpu.VMEM((_CH, 2 * H), jnp.float32),
            pltpu.SemaphoreType.DMA,
            pltpu.SemaphoreType.DMA,
        ],
    )
    return f(ps, pd, src, dst)


def _stage_c_body(ea, eb, w2cat, b2cat, g):
    z = jax.nn.relu(ea[...] + eb[...])
    h = z @ w2cat[...] + b2cat[...]
    k = jax.nn.sigmoid(h[:, 3 * H:3 * H + 1])
    g[...] = h[:, 0:3 * H] * k


def _stage_c(ea, eb, w2cat, b2cat):
    blk = 512
    grid = E_TOT // blk
    espec = pl.BlockSpec((blk, 2 * H), lambda i: (i, 0))
    return pl.pallas_call(
        _stage_c_body,
        grid=(grid,),
        in_specs=[espec, espec,
                  pl.BlockSpec((2 * H, 4 * H), lambda i: (0, 0)),
                  pl.BlockSpec((1, 4 * H), lambda i: (0, 0))],
        out_specs=pl.BlockSpec((blk, 3 * H), lambda i: (i, 0)),
        out_shape=jax.ShapeDtypeStruct((E_TOT, 3 * H), jnp.float32),
    )(ea, eb, w2cat, b2cat)


def _stage_e_body(gcx, msc, a2c, a3c, gsx, mss, a2s, a3s,
                  rwc0, rwc2, rwc3, rbc, rws0, rws2, rws3, rbs,
                  cw1a, cw1b, cb1, cw2, cb2, sw1a, sw1b, sb1, sw2, sb2,
                  out_fc, out_fs):
    new_cx = gcx[...] @ rwc0[...] + msc[...] + a2c[...] @ rwc2[...] \
        + a3c[...] @ rwc3[...] + rbc[...]
    t = jax.nn.relu(gcx[...] @ cw1a[...] + new_cx @ cw1b[...] + cb1[...])
    out_fc[...] = t @ cw2[...] + cb2[...]
    new_sx = gsx[...] @ rws0[...] + mss[...] + a2s[...] @ rws2[...] \
        + a3s[...] @ rws3[...] + rbs[...]
    u = jax.nn.relu(gsx[...] @ sw1a[...] + new_sx @ sw1b[...] + sb1[...])
    out_fs[...] = u @ sw2[...] + sb2[...]


def _stage_e(gc_x, msc, a2c, a3c, gs_x, mss, a2s, a3s,
             red_s2c_w, red_s2c_b, red_c2s_w, red_c2s_b,
             gc_w1, gc_b1, gc_w2, gc_b2, gs_w1, gs_b1, gs_w2, gs_b2):
    n = gc_x.shape[0]
    blk = 1000
    grid = n // blk
    row = pl.BlockSpec((blk, H), lambda i: (i, 0))
    wfull = pl.BlockSpec((H, H), lambda i: (0, 0))
    bfull = pl.BlockSpec((1, H), lambda i: (0, 0))
    return pl.pallas_call(
        _stage_e_body,
        grid=(grid,),
        in_specs=[row] * 8 + [wfull, wfull, wfull, bfull] * 2
        + [wfull, wfull, bfull, wfull, bfull] * 2,
        out_specs=(row, row),
        out_shape=(jax.ShapeDtypeStruct((n, H), jnp.float32),
                   jax.ShapeDtypeStruct((n, H), jnp.float32)),
    )(gc_x, msc, a2c, a3c, gs_x, mss, a2s, a3s,
      red_s2c_w[0:H], red_s2c_w[2 * H:3 * H], red_s2c_w[3 * H:4 * H],
      red_s2c_b.reshape(1, H),
      red_c2s_w[0:H], red_c2s_w[2 * H:3 * H], red_c2s_w[3 * H:4 * H],
      red_c2s_b.reshape(1, H),
      gc_w1[0:H], gc_w1[H:2 * H], gc_b1.reshape(1, H), gc_w2,
      gc_b2.reshape(1, H),
      gs_w1[0:H], gs_w1[H:2 * H], gs_b1.reshape(1, H), gs_w2,
      gs_b2.reshape(1, H))


def _edge_payload(ps, pd, src, dst, w2cat, b2cat):
    ea, eb = _sc_gather(ps, pd, src, dst)
    return _stage_c(ea, eb, w2cat, b2cat)


def kernel(nf_gc, nf_gs, ei_s2c, ei_c2s, lin_gc_w, lin_gc_b, lin_gs_w, lin_gs_b,
           msg_s2c_w1, msg_s2c_b1, msg_s2c_w2, msg_s2c_b2,
           red_s2c_w, red_s2c_b,
           msg_c2s_w1, msg_c2s_b1, msg_c2s_w2, msg_c2s_b2,
           red_c2s_w, red_c2s_b,
           gc_w1, gc_b1, gc_w2, gc_b2,
           gs_w1, gs_b1, gs_w2, gs_b2):
    gc_x, gs_x, ps_s2c, pd_s2c, ps_c2s, pd_c2s = _stage_a(
        nf_gc, nf_gs, lin_gc_w, lin_gc_b, lin_gs_w, lin_gs_b,
        msg_s2c_w1, msg_s2c_b1, msg_c2s_w1, msg_c2s_b1)
    w2cat_c, b2cat_c = _prep(msg_s2c_w2, msg_s2c_b2, red_s2c_w)
    w2cat_s, b2cat_s = _prep(msg_c2s_w2, msg_c2s_b2, red_c2s_w)

    g_c = _edge_payload(ps_s2c, pd_s2c, ei_s2c[0], ei_s2c[1], w2cat_c, b2cat_c)
    g_s = _edge_payload(ps_c2s, pd_c2s, ei_c2s[0], ei_c2s[1], w2cat_s, b2cat_s)

    dst_c = ei_s2c[1]
    dst_s = ei_c2s[1]
    msc = jax.ops.segment_sum(g_c[:, 0:H], dst_c, num_segments=N_NODE)
    a2c = jax.ops.segment_max(g_c[:, H:2 * H], dst_c, num_segments=N_NODE)
    a3c = jax.ops.segment_min(g_c[:, 2 * H:3 * H], dst_c, num_segments=N_NODE)
    mss = jax.ops.segment_sum(g_s[:, 0:H], dst_s, num_segments=N_NODE)
    a2s = jax.ops.segment_max(g_s[:, H:2 * H], dst_s, num_segments=N_NODE)
    a3s = jax.ops.segment_min(g_s[:, 2 * H:3 * H], dst_s, num_segments=N_NODE)

    return _stage_e(gc_x, msc, a2c, a3c, gs_x, mss, a2s, a3s,
                    red_s2c_w, red_s2c_b, red_c2s_w, red_c2s_b,
                    gc_w1, gc_b1, gc_w2, gc_b2, gs_w1, gs_b1, gs_w2, gs_b2)
